# baseline (speedup 1.0000x reference)
import functools

import jax
import jax.numpy as jnp
from jax import lax
from jax.experimental import pallas as pl
from jax.experimental.pallas import tpu as pltpu
from jax.experimental.pallas import tpu_sc as plsc

_LP6 = -0.017029610589052675
_LP5 = 0.08152317761736225
_LP4 = -0.18901954822291905
_LP3 = 0.31504127990864345
_LP2 = -0.49720333122019134
_LP1 = 0.9998325947816316
_LP0 = 1.6936626598407223e-06

_NC, _NS, _L = 2, 16, 16
_NW = _NC * _NS
_ACCW = 256


def _argmax_body(lab_ref, cls_ref):
    cls_ref[...] = jnp.argmax(lab_ref[...], axis=1).astype(jnp.int32)


def _finish_body(p_ref, o_ref):
    p = p_ref[...]
    cs = jnp.sum(p, axis=0)
    s_a = cs[0:64]
    cnt = cs[64:128]
    s_b = cs[128:192]
    c_b = jnp.sum(p[_NW // 2:, 64:128], axis=0)
    loss_a = jnp.mean(s_a / jnp.maximum(cnt, 1.0))
    loss_b = jnp.mean(s_b / jnp.maximum(c_b, 1.0))
    rr = lax.broadcasted_iota(jnp.int32, o_ref.shape, 0)
    ii = lax.broadcasted_iota(jnp.int32, o_ref.shape, 1)
    o_ref[...] = jnp.where((rr == 0) & (ii == 0), loss_a,
                           jnp.where((rr == 0) & (ii == 1), loss_b, 0.0))


def _make_sc_main(B, SPT):
    NROW = 2 * SPT
    NDMA = NROW // 128
    mesh = plsc.VectorSubcoreMesh(core_axis_name="c", subcore_axis_name="s")

    @functools.partial(
        pl.kernel,
        out_type=jax.ShapeDtypeStruct((_NW, _ACCW), jnp.float32),
        mesh=mesh,
        compiler_params=pltpu.CompilerParams(needs_layout_passes=False,
                                             use_tc_tiling_on_sc=False),
        scratch_types=[
            pltpu.VMEM((SPT,), jnp.int32),
            pltpu.VMEM((NDMA, 128), jnp.int32),
            pltpu.VMEM((NROW, 8), jnp.float32),
            pltpu.VMEM((NROW,), jnp.float32),
            pltpu.VMEM((_L * _ACCW,), jnp.float32),
            pltpu.VMEM((_ACCW,), jnp.float32),
            pltpu.SemaphoreType.DMA,
        ],
    )
    def sc_main(nat8_hbm, cls_hbm, dom_hbm, out_hbm,
                cls_v, idx_v, buf_v, dom_v, acc_v, part_v, sem):
        wid = lax.axis_index("s") * _NC + lax.axis_index("c")
        base = wid * SPT
        lane = lax.iota(jnp.int32, 16)
        ones = jnp.ones((16,), jnp.float32)
        twos = ones + ones
        lbase = lane * _ACCW
        is_tgt = wid >= (_NW // 2)

        pltpu.sync_copy(cls_hbm.at[pl.ds(base, SPT)], cls_v)
        pltpu.sync_copy(dom_hbm.at[pl.ds(2 * base, NROW)], dom_v)

        zz = jnp.zeros((16,), jnp.float32)
        for i in range((_L * _ACCW) // 16):
            acc_v[pl.ds(i * 16, 16)] = zz

        half = SPT // 128
        for i in range(SPT // 16):
            k0 = i * 16
            clsr = cls_v[pl.ds(k0, 16)]
            sg = base + k0 + lane
            r0 = clsr * 4096 + ((sg >> 7) << 5) + ((sg & 127) >> 3)
            cpos = (i & 7) * 16
            idx_v[i >> 3, pl.ds(cpos, 16)] = r0
            idx_v[half + (i >> 3), pl.ds(cpos, 16)] = r0 + 16
            plsc.addupdate_scatter(acc_v, [lbase + 64 + clsr], twos)

        copies = [
            pltpu.async_copy(nat8_hbm.at[idx_v.at[q]],
                             buf_v.at[pl.ds(q * 128, 128)], sem)
            for q in range(NDMA)
        ]
        for cp in copies:
            cp.wait()

        j16 = lane & 1
        for i in range(NROW // 16):
            k0 = i * 16
            s16 = (k0 + lane) >> 1
            x = plsc.load_gather(buf_v, [j16 * SPT + s16, s16 & 7])
            y = plsc.load_gather(dom_v, [((s16 >> 7) << 8) + (j16 << 7)
                                         + (s16 & 127)])
            clsr = plsc.load_gather(cls_v, [s16])
            t = jnp.exp(-jnp.abs(x))
            lp = ((((((_LP6 * t + _LP5) * t + _LP4) * t + _LP3) * t + _LP2)
                   * t + _LP1) * t + _LP0)
            b_a = jnp.maximum(x, 0.0) - x * y + lp
            slot = lbase + clsr
            plsc.addupdate_scatter(acc_v, [slot], b_a)

            @pl.when(is_tgt)
            def _():
                b_b = b_a + x * (2.0 * y - 1.0)
                plsc.addupdate_scatter(acc_v, [slot + 128], b_b)

        for j in range(_ACCW // 16):
            o = j * 16
            s = acc_v[pl.ds(o, 16)]
            for l in range(1, _L):
                s = s + acc_v[pl.ds(l * _ACCW + o, 16)]
            part_v[pl.ds(o, 16)] = s
        pltpu.sync_copy(part_v, out_hbm.at[wid])

    return sc_main


def kernel(logits_list, labels, domain, target_start_id):
    del target_start_id
    C, B, _ = logits_list.shape
    SPT = B // _NW

    nblk = 8
    cls = pl.pallas_call(
        _argmax_body,
        grid=(nblk,),
        in_specs=[pl.BlockSpec((B // nblk, C), lambda i: (i, 0))],
        out_specs=pl.BlockSpec((B // nblk,), lambda i: (i,)),
        out_shape=jax.ShapeDtypeStruct((B,), jnp.int32),
    )(labels)

    nat8 = (logits_list.reshape(C, B // 128, 128, 2)
            .transpose(0, 1, 3, 2).reshape(C * B * 2 // 8, 8))
    dom_nat = (domain.reshape(B // 128, 128, 2)
               .transpose(0, 2, 1).reshape(2 * B))
    partials = _make_sc_main(B, SPT)(nat8, cls, dom_nat)

    out = pl.pallas_call(
        _finish_body,
        in_specs=[pl.BlockSpec((_NW, _ACCW), lambda: (0, 0))],
        out_specs=pl.BlockSpec((8, 128), lambda: (0, 0)),
        out_shape=jax.ShapeDtypeStruct((8, 128), jnp.float32),
    )(partials)
    return (out[0, 0], out[0, 1])


# speedup vs baseline: 1.1644x; 1.1644x over previous
import functools

import jax
import jax.numpy as jnp
from jax import lax
from jax.experimental import pallas as pl
from jax.experimental.pallas import tpu as pltpu
from jax.experimental.pallas import tpu_sc as plsc

_LP6 = -0.017029610589052675
_LP5 = 0.08152317761736225
_LP4 = -0.18901954822291905
_LP3 = 0.31504127990864345
_LP2 = -0.49720333122019134
_LP1 = 0.9998325947816316
_LP0 = 1.6936626598407223e-06

_NC, _NS, _L = 2, 16, 16
_NW = _NC * _NS
_ACCW = 256


def _argmax_body(lab_ref, cls_ref):
    cls_ref[...] = jnp.argmax(lab_ref[...], axis=1).astype(jnp.int32)


def _finish_body(p_ref, o_ref):
    p = p_ref[...]
    cs = jnp.sum(p, axis=0)
    s_a = cs[0:64]
    cnt = cs[64:128]
    s_b = cs[128:192]
    c_b = jnp.sum(p[_NW // 2:, 64:128], axis=0)
    loss_a = jnp.mean(s_a / jnp.maximum(cnt, 1.0))
    loss_b = jnp.mean(s_b / jnp.maximum(c_b, 1.0))
    rr = lax.broadcasted_iota(jnp.int32, o_ref.shape, 0)
    ii = lax.broadcasted_iota(jnp.int32, o_ref.shape, 1)
    o_ref[...] = jnp.where((rr == 0) & (ii == 0), loss_a,
                           jnp.where((rr == 0) & (ii == 1), loss_b, 0.0))


def _make_sc_main(B, SPT):
    NROW = 2 * SPT
    NDMA = NROW // 128
    mesh = plsc.VectorSubcoreMesh(core_axis_name="c", subcore_axis_name="s")

    @functools.partial(
        pl.kernel,
        out_type=jax.ShapeDtypeStruct((_NW, _ACCW), jnp.float32),
        mesh=mesh,
        compiler_params=pltpu.CompilerParams(needs_layout_passes=False,
                                             use_tc_tiling_on_sc=False),
        scratch_types=[
            pltpu.VMEM((SPT,), jnp.int32),
            pltpu.VMEM((NDMA, 128), jnp.int32),
            pltpu.VMEM((NROW, 8), jnp.float32),
            pltpu.VMEM((NROW,), jnp.float32),
            pltpu.VMEM((_L * _ACCW,), jnp.float32),
            pltpu.VMEM((_ACCW,), jnp.float32),
            pltpu.SemaphoreType.DMA,
        ],
    )
    def sc_main(nat8_hbm, cls_hbm, dom_hbm, out_hbm,
                cls_v, idx_v, buf_v, dom_v, acc_v, part_v, sem):
        wid = lax.axis_index("s") * _NC + lax.axis_index("c")
        base = wid * SPT
        lane = lax.iota(jnp.int32, 16)
        ones = jnp.ones((16,), jnp.float32)
        twos = ones + ones
        lbase = lane * _ACCW
        is_tgt = wid >= (_NW // 2)

        pltpu.sync_copy(cls_hbm.at[pl.ds(base, SPT)], cls_v)
        pltpu.sync_copy(dom_hbm.at[pl.ds(2 * base, NROW)], dom_v)

        zz = jnp.zeros((16,), jnp.float32)

        def zero_loop(i, c):
            for u in range(4):
                acc_v[pl.ds((i * 4 + u) * 16, 16)] = zz
            return c

        lax.fori_loop(0, (_L * _ACCW) // 64, zero_loop, 0)

        half = SPT // 128

        def build(i8, c):
            for u in range(2):
                i = i8 * 2 + u
                k0 = i * 16
                clsr = cls_v[pl.ds(k0, 16)]
                sg = base + k0 + lane
                r0 = clsr * 4096 + ((sg >> 7) << 5) + ((sg & 127) >> 3)
                cpos = (i & 7) * 16
                idx_v[i >> 3, pl.ds(cpos, 16)] = r0
                idx_v[half + (i >> 3), pl.ds(cpos, 16)] = r0 + 16
                plsc.addupdate_scatter(acc_v, [lbase + 64 + clsr], twos)
            return c

        lax.fori_loop(0, SPT // 32, build, 0)

        copies = [
            pltpu.async_copy(nat8_hbm.at[idx_v.at[q]],
                             buf_v.at[pl.ds(q * 128, 128)], sem)
            for q in range(NDMA)
        ]
        for cp in copies:
            cp.wait()

        j16 = lane & 1

        def bce(i2, c):
            for u in range(2):
                k0 = (i2 * 2 + u) * 16
                s16 = (k0 + lane) >> 1
                x = plsc.load_gather(buf_v, [j16 * SPT + s16, s16 & 7])
                y = plsc.load_gather(dom_v, [((s16 >> 7) << 8) + (j16 << 7)
                                             + (s16 & 127)])
                clsr = plsc.load_gather(cls_v, [s16])
                t = jnp.exp(-jnp.abs(x))
                lp = ((((((_LP6 * t + _LP5) * t + _LP4) * t + _LP3) * t
                        + _LP2) * t + _LP1) * t + _LP0)
                b_a = jnp.maximum(x, 0.0) - x * y + lp
                slot = lbase + clsr
                plsc.addupdate_scatter(acc_v, [slot], b_a)

                @pl.when(is_tgt)
                def _():
                    b_b = b_a + x * (2.0 * y - 1.0)
                    plsc.addupdate_scatter(acc_v, [slot + 128], b_b)

            return c

        lax.fori_loop(0, NROW // 32, bce, 0)

        def red(j, c):
            o = j * 16
            s = acc_v[pl.ds(o, 16)]
            for l in range(1, _L):
                s = s + acc_v[pl.ds(l * _ACCW + o, 16)]
            part_v[pl.ds(o, 16)] = s
            return c

        lax.fori_loop(0, _ACCW // 16, red, 0)
        pltpu.sync_copy(part_v, out_hbm.at[wid])

    return sc_main


def kernel(logits_list, labels, domain, target_start_id):
    del target_start_id
    C, B, _ = logits_list.shape
    SPT = B // _NW

    nblk = 8
    cls = pl.pallas_call(
        _argmax_body,
        grid=(nblk,),
        in_specs=[pl.BlockSpec((B // nblk, C), lambda i: (i, 0))],
        out_specs=pl.BlockSpec((B // nblk,), lambda i: (i,)),
        out_shape=jax.ShapeDtypeStruct((B,), jnp.int32),
    )(labels)

    nat8 = (logits_list.reshape(C, B // 128, 128, 2)
            .transpose(0, 1, 3, 2).reshape(C * B * 2 // 8, 8))
    dom_nat = (domain.reshape(B // 128, 128, 2)
               .transpose(0, 2, 1).reshape(2 * B))
    partials = _make_sc_main(B, SPT)(nat8, cls, dom_nat)

    out = pl.pallas_call(
        _finish_body,
        in_specs=[pl.BlockSpec((_NW, _ACCW), lambda: (0, 0))],
        out_specs=pl.BlockSpec((8, 128), lambda: (0, 0)),
        out_shape=jax.ShapeDtypeStruct((8, 128), jnp.float32),
    )(partials)
    return (out[0, 0], out[0, 1])


# speedup vs baseline: 1.2199x; 1.0476x over previous
import functools

import jax
import jax.numpy as jnp
from jax import lax
from jax.experimental import pallas as pl
from jax.experimental.pallas import tpu as pltpu
from jax.experimental.pallas import tpu_sc as plsc

_LP6 = -0.017029610589052675
_LP5 = 0.08152317761736225
_LP4 = -0.18901954822291905
_LP3 = 0.31504127990864345
_LP2 = -0.49720333122019134
_LP1 = 0.9998325947816316
_LP0 = 1.6936626598407223e-06

_NC, _NS, _L = 2, 16, 16
_NW = _NC * _NS
_ACCW = 256


def _argmax_body(lab_ref, cls_ref):
    cls_ref[...] = jnp.argmax(lab_ref[...], axis=1).astype(jnp.int32)


def _finish_body(p_ref, o_ref):
    p = p_ref[...]
    cs = jnp.sum(p, axis=0)
    s_a = cs[0:64]
    cnt = cs[64:128]
    s_b = cs[128:192]
    c_b = jnp.sum(p[_NW // 2:, 64:128], axis=0)
    loss_a = jnp.mean(s_a / jnp.maximum(cnt, 1.0))
    loss_b = jnp.mean(s_b / jnp.maximum(c_b, 1.0))
    rr = lax.broadcasted_iota(jnp.int32, o_ref.shape, 0)
    ii = lax.broadcasted_iota(jnp.int32, o_ref.shape, 1)
    o_ref[...] = jnp.where((rr == 0) & (ii == 0), loss_a,
                           jnp.where((rr == 0) & (ii == 1), loss_b, 0.0))


def _make_sc_main(B, SPT):
    NROW = 2 * SPT
    NDMA = NROW // 128
    mesh = plsc.VectorSubcoreMesh(core_axis_name="c", subcore_axis_name="s")

    @functools.partial(
        pl.kernel,
        out_type=jax.ShapeDtypeStruct((_NW, _ACCW), jnp.float32),
        mesh=mesh,
        compiler_params=pltpu.CompilerParams(needs_layout_passes=False,
                                             use_tc_tiling_on_sc=False),
        scratch_types=[
            pltpu.VMEM((SPT,), jnp.int32),
            pltpu.VMEM((NDMA, 128), jnp.int32),
            pltpu.VMEM((NROW, 8), jnp.float32),
            pltpu.VMEM((NROW,), jnp.float32),
            pltpu.VMEM((_L * _ACCW,), jnp.float32),
            pltpu.VMEM((_ACCW,), jnp.float32),
            pltpu.SemaphoreType.DMA,
        ],
    )
    def sc_main(nat8_hbm, cls_hbm, dom_hbm, out_hbm,
                cls_v, idx_v, buf_v, dom_v, acc_v, part_v, sem):
        wid = lax.axis_index("s") * _NC + lax.axis_index("c")
        base = wid * SPT
        lane = lax.iota(jnp.int32, 16)
        ones = jnp.ones((16,), jnp.float32)
        lbase = lane * _ACCW
        is_tgt = wid >= (_NW // 2)

        pltpu.sync_copy(cls_hbm.at[pl.ds(base, SPT)], cls_v)

        half = SPT // 128

        def build(i8, c):
            for u in range(2):
                i = i8 * 2 + u
                k0 = i * 16
                clsr = cls_v[pl.ds(k0, 16)]
                sg = base + k0 + lane
                r0 = clsr * 4096 + ((sg >> 7) << 5) + ((sg & 127) >> 3)
                cpos = (i & 7) * 16
                idx_v[i >> 3, pl.ds(cpos, 16)] = r0
                idx_v[half + (i >> 3), pl.ds(cpos, 16)] = r0 + 16
            return c

        lax.fori_loop(0, SPT // 32, build, 0)

        copies = [
            pltpu.async_copy(nat8_hbm.at[idx_v.at[q]],
                             buf_v.at[pl.ds(q * 128, 128)], sem)
            for q in range(NDMA)
        ]

        pltpu.sync_copy(dom_hbm.at[pl.ds(2 * base, NROW)], dom_v)
        zz = jnp.zeros((16,), jnp.float32)

        def zero_loop(i, c):
            for u in range(4):
                acc_v[pl.ds((i * 4 + u) * 16, 16)] = zz
            return c

        lax.fori_loop(0, (_L * _ACCW) // 64, zero_loop, 0)

        for cp in copies:
            cp.wait()

        j16 = lane & 1

        def bce(i2, c):
            for u in range(2):
                k0 = (i2 * 2 + u) * 16
                s16 = (k0 + lane) >> 1
                x = plsc.load_gather(buf_v, [j16 * SPT + s16, s16 & 7])
                y = plsc.load_gather(dom_v, [((s16 >> 7) << 8) + (j16 << 7)
                                             + (s16 & 127)])
                clsr = plsc.load_gather(cls_v, [s16])
                t = jnp.exp(-jnp.abs(x))
                lp = ((((((_LP6 * t + _LP5) * t + _LP4) * t + _LP3) * t
                        + _LP2) * t + _LP1) * t + _LP0)
                b_a = jnp.maximum(x, 0.0) - x * y + lp
                slot = lbase + clsr
                plsc.addupdate_scatter(acc_v, [slot], b_a)
                plsc.addupdate_scatter(acc_v, [slot + 64], ones)

                @pl.when(is_tgt)
                def _():
                    b_b = b_a + x * (2.0 * y - 1.0)
                    plsc.addupdate_scatter(acc_v, [slot + 128], b_b)

            return c

        lax.fori_loop(0, NROW // 32, bce, 0)

        def red(j, c):
            o = j * 16
            s = acc_v[pl.ds(o, 16)]
            for l in range(1, _L):
                s = s + acc_v[pl.ds(l * _ACCW + o, 16)]
            part_v[pl.ds(o, 16)] = s
            return c

        lax.fori_loop(0, _ACCW // 16, red, 0)
        pltpu.sync_copy(part_v, out_hbm.at[wid])

    return sc_main


def kernel(logits_list, labels, domain, target_start_id):
    del target_start_id
    C, B, _ = logits_list.shape
    SPT = B // _NW

    nblk = 4
    cls = pl.pallas_call(
        _argmax_body,
        grid=(nblk,),
        in_specs=[pl.BlockSpec((B // nblk, C), lambda i: (i, 0))],
        out_specs=pl.BlockSpec((B // nblk,), lambda i: (i,)),
        out_shape=jax.ShapeDtypeStruct((B,), jnp.int32),
    )(labels)

    nat8 = (logits_list.reshape(C, B // 128, 128, 2)
            .transpose(0, 1, 3, 2).reshape(C * B * 2 // 8, 8))
    dom_nat = (domain.reshape(B // 128, 128, 2)
               .transpose(0, 2, 1).reshape(2 * B))
    partials = _make_sc_main(B, SPT)(nat8, cls, dom_nat)

    out = pl.pallas_call(
        _finish_body,
        in_specs=[pl.BlockSpec((_NW, _ACCW), lambda: (0, 0))],
        out_specs=pl.BlockSpec((8, 128), lambda: (0, 0)),
        out_shape=jax.ShapeDtypeStruct((8, 128), jnp.float32),
    )(partials)
    return (out[0, 0], out[0, 1])
